# Initial kernel scaffold; baseline (speedup 1.0000x reference)
import jax
import jax.numpy as jnp
from jax.experimental import pallas as pl


def kernel(x, scale, gate_kernel, gate_bias, mlp1_weight, mlp1_bias, mlp2_weight, mlp2_bias):
    raise NotImplementedError("write your pallas kernel here")


# speedup vs baseline: 3.6501x; 3.6501x over previous
import jax
import jax.numpy as jnp
from jax.experimental import pallas as pl
from jax.experimental.pallas import tpu as pltpu

N_TOKENS = 64
HIDDEN = 1024
INTER = 1024
N_EXPERTS = 16
TOP_K = 2
SWIGLU_LIMIT = 7.0
EPS = 1e-05
ALPHA = 1.702


def _moe_kernel(xt_ref, scale_ref, eit_ref, elt_ref, w1_ref, b1_ref, w2_ref,
                b2_ref, out_ref, t2_ref, wt_ref):
    e = pl.program_id(0)
    sidx = jax.lax.broadcasted_iota(jnp.int32, (N_EXPERTS, N_TOKENS), 0)

    @pl.when(e == 0)
    def _prologue():
        xv = xt_ref[...]
        ms = jnp.mean(xv * xv, axis=0, keepdims=True)
        tn = ((xv / jnp.sqrt(ms + EPS)) * scale_ref[...]).astype(jnp.bfloat16)
        t2_ref[...] = jnp.zeros((2 * HIDDEN, 2 * N_TOKENS), jnp.bfloat16)
        t2_ref[0:HIDDEN, 0:N_TOKENS] = tn
        t2_ref[HIDDEN:2 * HIDDEN, N_TOKENS:2 * N_TOKENS] = tn
        i1 = eit_ref[0:1, :]
        i2 = eit_ref[1:2, :]
        l1 = elt_ref[0:1, :]
        l2 = elt_ref[1:2, :]
        e2 = jnp.exp(l2 - l1)
        denom = 1.0 + e2
        wt_ref[...] = (jnp.where(sidx == i1, 1.0 / denom, 0.0)
                       + jnp.where(sidx == i2, e2 / denom, 0.0))
        out_ref[...] = xv

    h2 = jax.lax.dot_general(w1_ref[0], t2_ref[...], (((1,), (0,)), ((), ())),
                             preferred_element_type=jnp.float32)
    b1v = b1_ref[0]
    hg = (h2[:, 0:N_TOKENS] + b1v[:, 0:1]).astype(jnp.bfloat16)
    hl = (h2[:, N_TOKENS:2 * N_TOKENS] + b1v[:, 1:2]).astype(jnp.bfloat16)
    glu = jnp.minimum(hg, SWIGLU_LIMIT)
    lin = jnp.clip(hl, -SWIGLU_LIMIT, SWIGLU_LIMIT)
    act = (glu * jax.nn.sigmoid(ALPHA * glu)) * (lin + 1.0)
    y = jax.lax.dot_general(w2_ref[0], act, (((1,), (0,)), ((), ())),
                            preferred_element_type=jnp.float32)
    y = y + b2_ref[0]
    wrow = jnp.sum(jnp.where(sidx == e, wt_ref[...], 0.0), axis=0,
                   keepdims=True)
    out_ref[...] += y * wrow


@jax.jit
def kernel(x, scale, gate_kernel, gate_bias, mlp1_weight, mlp1_bias,
           mlp2_weight, mlp2_bias):
    t = x.astype(jnp.float32)
    rms = jnp.sqrt(jnp.mean(t ** 2, axis=-1, keepdims=True) + EPS)
    t = ((t / rms) * scale).astype(jnp.bfloat16)
    g = jnp.matmul(t, gate_kernel) + gate_bias
    expert_logits, expert_indices = jax.lax.top_k(g, TOP_K)

    out_t = pl.pallas_call(
        _moe_kernel,
        grid=(N_EXPERTS,),
        in_specs=[
            pl.BlockSpec((HIDDEN, N_TOKENS), lambda e: (0, 0)),
            pl.BlockSpec((HIDDEN, 1), lambda e: (0, 0)),
            pl.BlockSpec((TOP_K, N_TOKENS), lambda e: (0, 0)),
            pl.BlockSpec((TOP_K, N_TOKENS), lambda e: (0, 0)),
            pl.BlockSpec((1, INTER, 2 * HIDDEN), lambda e: (e, 0, 0)),
            pl.BlockSpec((1, INTER, 2), lambda e: (e, 0, 0)),
            pl.BlockSpec((1, HIDDEN, INTER), lambda e: (e, 0, 0)),
            pl.BlockSpec((1, HIDDEN, 1), lambda e: (e, 0, 0)),
        ],
        out_specs=pl.BlockSpec((HIDDEN, N_TOKENS), lambda e: (0, 0)),
        out_shape=jax.ShapeDtypeStruct((HIDDEN, N_TOKENS), jnp.float32),
        scratch_shapes=[
            pltpu.VMEM((2 * HIDDEN, 2 * N_TOKENS), jnp.bfloat16),
            pltpu.VMEM((N_EXPERTS, N_TOKENS), jnp.float32),
        ],
    )(
        x.T,
        scale.reshape(HIDDEN, 1),
        expert_indices.T.astype(jnp.int32),
        expert_logits.T.astype(jnp.float32),
        mlp1_weight.reshape(N_EXPERTS, INTER, 2 * HIDDEN),
        mlp1_bias.reshape(N_EXPERTS, INTER, 2),
        mlp2_weight,
        mlp2_bias.reshape(N_EXPERTS, HIDDEN, 1),
    )
    return out_t.T


# speedup vs baseline: 6.4755x; 1.7741x over previous
import jax
import jax.numpy as jnp
from jax.experimental import pallas as pl
from jax.experimental.pallas import tpu as pltpu

N_TOKENS = 64
HIDDEN = 1024
INTER = 1024
N_EXPERTS = 16
TOP_K = 2
SWIGLU_LIMIT = 7.0
EPS = 1e-05
ALPHA = 1.702


def _moe_kernel(xt_ref, scale_ref, eit_ref, elt_ref, w1_ref, b1_ref, w2_ref,
                b2_ref, out_ref, tn_ref, wt_ref):
    e = pl.program_id(0)
    sidx = jax.lax.broadcasted_iota(jnp.int32, (N_EXPERTS, N_TOKENS), 0)

    @pl.when(e == 0)
    def _prologue():
        xv = xt_ref[...]
        ms = jnp.mean(xv * xv, axis=0, keepdims=True)
        tn_ref[...] = ((xv / jnp.sqrt(ms + EPS))
                       * scale_ref[...]).astype(jnp.bfloat16)
        i1 = eit_ref[0:1, :]
        i2 = eit_ref[1:2, :]
        l1 = elt_ref[0:1, :]
        l2 = elt_ref[1:2, :]
        e2 = jnp.exp(l2 - l1)
        denom = 1.0 + e2
        wt_ref[...] = (jnp.where(sidx == i1, 1.0 / denom, 0.0)
                       + jnp.where(sidx == i2, e2 / denom, 0.0))
        out_ref[...] = xv

    h = jax.lax.dot_general(w1_ref[0], tn_ref[...], (((1,), (0,)), ((), ())),
                            preferred_element_type=jnp.float32)
    h = h + b1_ref[0]
    h3 = h.reshape(INTER, 2, N_TOKENS)
    hg = h3[:, 0, :].astype(jnp.bfloat16)
    hl = h3[:, 1, :].astype(jnp.bfloat16)
    glu = jnp.minimum(hg, SWIGLU_LIMIT)
    lin = jnp.clip(hl, -SWIGLU_LIMIT, SWIGLU_LIMIT)
    act = (glu * jax.nn.sigmoid(ALPHA * glu)) * (lin + 1.0)
    y = jax.lax.dot_general(w2_ref[0], act, (((1,), (0,)), ((), ())),
                            preferred_element_type=jnp.float32)
    y = y + b2_ref[0]
    wrow = jnp.sum(jnp.where(sidx == e, wt_ref[...], 0.0), axis=0,
                   keepdims=True)
    out_ref[...] += y * wrow


@jax.jit
def kernel(x, scale, gate_kernel, gate_bias, mlp1_weight, mlp1_bias,
           mlp2_weight, mlp2_bias):
    t = x.astype(jnp.float32)
    rms = jnp.sqrt(jnp.mean(t ** 2, axis=-1, keepdims=True) + EPS)
    t = ((t / rms) * scale).astype(jnp.bfloat16)
    g = jnp.matmul(t, gate_kernel) + gate_bias
    expert_logits, expert_indices = jax.lax.top_k(g, TOP_K)

    out_t = pl.pallas_call(
        _moe_kernel,
        grid=(N_EXPERTS,),
        in_specs=[
            pl.BlockSpec((HIDDEN, N_TOKENS), lambda e: (0, 0)),
            pl.BlockSpec((HIDDEN, 1), lambda e: (0, 0)),
            pl.BlockSpec((TOP_K, N_TOKENS), lambda e: (0, 0)),
            pl.BlockSpec((TOP_K, N_TOKENS), lambda e: (0, 0)),
            pl.BlockSpec((1, 2 * INTER, HIDDEN), lambda e: (e, 0, 0)),
            pl.BlockSpec((1, 2 * INTER, 1), lambda e: (e, 0, 0)),
            pl.BlockSpec((1, HIDDEN, INTER), lambda e: (e, 0, 0)),
            pl.BlockSpec((1, HIDDEN, 1), lambda e: (e, 0, 0)),
        ],
        out_specs=pl.BlockSpec((HIDDEN, N_TOKENS), lambda e: (0, 0)),
        out_shape=jax.ShapeDtypeStruct((HIDDEN, N_TOKENS), jnp.float32),
        scratch_shapes=[
            pltpu.VMEM((HIDDEN, N_TOKENS), jnp.bfloat16),
            pltpu.VMEM((N_EXPERTS, N_TOKENS), jnp.float32),
        ],
    )(
        x.T,
        scale.reshape(HIDDEN, 1),
        expert_indices.T.astype(jnp.int32),
        expert_logits.T.astype(jnp.float32),
        mlp1_weight,
        mlp1_bias.astype(jnp.float32).reshape(N_EXPERTS, 2 * INTER, 1),
        mlp2_weight,
        mlp2_bias.reshape(N_EXPERTS, HIDDEN, 1),
    )
    return out_t.T
